# Initial kernel scaffold; baseline (speedup 1.0000x reference)
import jax
import jax.numpy as jnp
from jax.experimental import pallas as pl


def kernel(x, edge_index, batch, Wl1, Wr1, att1, b1, Wl2, Wr2, att2, b2, g1W, g1b, g2W, g2b, l1W, l1b, l2W, l2b):
    raise NotImplementedError("write your pallas kernel here")


# speedup vs baseline: 24.8180x; 24.8180x over previous
import functools

import jax
import jax.numpy as jnp
from jax import lax
from jax.experimental import pallas as pl
from jax.experimental.pallas import tpu as pltpu
from jax.experimental.pallas import tpu_sc as plsc

_L = 16
_C = 128
_NC = 2
_NS = 16
_F32 = jnp.float32


def _proj_body(x_ref, wl_ref, wr_ref, xl_ref, xr_ref):
    xb = x_ref[...]
    xl_ref[...] = jnp.dot(xb, wl_ref[...], preferred_element_type=_F32)
    xr_ref[...] = jnp.dot(xb, wr_ref[...], preferred_element_type=_F32)


def _project(x, wl, wr, blk=1000):
    n, k = x.shape
    d = wl.shape[1]
    grid = (n + blk - 1) // blk
    return pl.pallas_call(
        _proj_body,
        grid=(grid,),
        in_specs=[
            pl.BlockSpec((blk, k), lambda i: (i, 0)),
            pl.BlockSpec((k, d), lambda i: (0, 0)),
            pl.BlockSpec((k, d), lambda i: (0, 0)),
        ],
        out_specs=[
            pl.BlockSpec((blk, d), lambda i: (i, 0)),
            pl.BlockSpec((blk, d), lambda i: (i, 0)),
        ],
        out_shape=[
            jax.ShapeDtypeStruct((n, d), _F32),
            jax.ShapeDtypeStruct((n, d), _F32),
        ],
    )(x, wl, wr)


def _make_edge_kernel(n_nodes, d, heads, n_edges):
    ch = d // heads
    assert n_edges % _C == 0
    n_chunks = n_edges // _C
    assert n_chunks % _NC == 0
    per_core = n_chunks // _NC
    base_loc = per_core // _NS
    rem = per_core % _NS
    assert n_nodes % _NS == 0
    rps = (n_nodes // _NS) & ~7
    tail = n_nodes - rps * _NS
    assert tail % 8 == 0

    mesh = plsc.VectorSubcoreMesh(core_axis_name="c", subcore_axis_name="s")

    @functools.partial(
        pl.kernel,
        out_type=(
            jax.ShapeDtypeStruct((_NC, n_nodes, d), _F32),
            jax.ShapeDtypeStruct((_NC, n_nodes, heads), _F32),
        ),
        mesh=mesh,
        compiler_params=pltpu.CompilerParams(needs_layout_passes=False,
                                             use_tc_tiling_on_sc=False),
        scratch_types=[
            pltpu.VMEM((_C,), jnp.int32),
            pltpu.VMEM((_C,), jnp.int32),
            pltpu.VMEM((_C, d), _F32),
            pltpu.VMEM((_C, d), _F32),
            pltpu.VMEM((_C, d), _F32),
            pltpu.VMEM((_C, heads), _F32),
            pltpu.VMEM((d * _L,), _F32),
            pltpu.VMEM_SHARED((n_nodes, d), _F32),
            pltpu.VMEM_SHARED((n_nodes, heads), _F32),
            pltpu.SemaphoreType.DMA,
            pltpu.SemaphoreType.DMA,
        ],
    )
    def edge_kernel(xl_hbm, xr_hbm, src_hbm, dst_hbm, attb_hbm, znum_hbm,
                    zden_hbm, num_out, den_out, src_v, dst_v, xl_rows,
                    xr_rows, scaled, pbuf, att_v, num_acc, den_acc, sem0,
                    sem1):
        cid = lax.axis_index("c")
        sid = lax.axis_index("s")
        iota = lax.iota(jnp.int32, _L)

        pltpu.sync_copy(attb_hbm, att_v)

        r0 = sid * rps
        pltpu.sync_copy(znum_hbm.at[pl.ds(0, rps)], num_acc.at[pl.ds(r0, rps)])
        pltpu.sync_copy(zden_hbm.at[pl.ds(0, rps)], den_acc.at[pl.ds(r0, rps)])
        if tail:
            @pl.when(sid == _NS - 1)
            def _():
                t0 = rps * _NS
                pltpu.sync_copy(znum_hbm.at[pl.ds(0, tail)],
                                num_acc.at[pl.ds(t0, tail)])
                pltpu.sync_copy(zden_hbm.at[pl.ds(0, tail)],
                                den_acc.at[pl.ds(t0, tail)])
        plsc.subcore_barrier()

        nloc = base_loc + jnp.where(sid < rem, 1, 0)

        def chunk_body(i, carry):
            chunk = cid * per_core + sid + i * _NS
            base = chunk * _C
            cp0 = pltpu.async_copy(src_hbm.at[pl.ds(base, _C)], src_v, sem0)
            cp1 = pltpu.async_copy(dst_hbm.at[pl.ds(base, _C)], dst_v, sem1)
            cp0.wait()
            cp1.wait()
            g0 = pltpu.async_copy(xl_hbm.at[src_v], xl_rows, sem0)
            g1 = pltpu.async_copy(xr_hbm.at[dst_v], xr_rows, sem1)
            g0.wait()
            g1.wait()

            for h in range(heads):
                hvec = jnp.full((_L,), h, jnp.int32)

                @pl.loop(0, _C // _L)
                def _(g):
                    rows = g * _L + iota
                    acc = jnp.zeros((_L,), _F32)
                    avals = []
                    for c in range(h * ch, (h + 1) * ch):
                        cvec = jnp.full((_L,), c, jnp.int32)
                        a = plsc.load_gather(xl_rows, [rows, cvec])
                        b = plsc.load_gather(xr_rows, [rows, cvec])
                        m = a + b
                        m = jnp.where(m >= 0.0, m, 0.2 * m)
                        att = att_v[pl.ds(c * _L, _L)]
                        acc = acc + m * att
                        avals.append(a)
                    p = jnp.exp(acc)
                    plsc.store_scatter(pbuf, [rows, hvec], p)
                    for c in range(h * ch, (h + 1) * ch):
                        cvec = jnp.full((_L,), c, jnp.int32)
                        plsc.store_scatter(scaled, [rows, cvec],
                                           avals[c - h * ch] * p)

            pltpu.sync_copy(scaled, num_acc.at[dst_v], add=True)
            pltpu.sync_copy(pbuf, den_acc.at[dst_v], add=True)
            return carry

        lax.fori_loop(0, nloc, chunk_body, 0)

        plsc.subcore_barrier()
        pltpu.sync_copy(num_acc.at[pl.ds(r0, rps)],
                        num_out.at[cid, pl.ds(r0, rps)])
        pltpu.sync_copy(den_acc.at[pl.ds(r0, rps)],
                        den_out.at[cid, pl.ds(r0, rps)])
        if tail:
            @pl.when(sid == _NS - 1)
            def _():
                t0 = rps * _NS
                pltpu.sync_copy(num_acc.at[pl.ds(t0, tail)],
                                num_out.at[cid, pl.ds(t0, tail)])
                pltpu.sync_copy(den_acc.at[pl.ds(t0, tail)],
                                den_out.at[cid, pl.ds(t0, tail)])

    return edge_kernel


def _edge_sweep(xl, xr, src, dst, att):
    n, d = xl.shape
    heads = att.shape[0]
    e = src.shape[0]
    attb = jnp.broadcast_to(att.reshape(d, 1), (d, _L)).reshape(d * _L)
    znum = jnp.zeros((n // _NS, d), _F32)
    zden = jnp.zeros((n // _NS, heads), _F32)
    k = _make_edge_kernel(n, d, heads, e)
    num, den = k(xl, xr, src, dst, attb, znum, zden)
    return num, den


def _make_norm_body(heads, ch):
    def body(num_ref, den_ref, b_ref, wl_ref, wr_ref, xl_ref, xr_ref):
        n = num_ref[0] + num_ref[1]
        dsum = den_ref[0] + den_ref[1]
        parts = [
            n[:, h * ch:(h + 1) * ch] / (dsum[:, h:h + 1] + 1e-16)
            for h in range(heads)
        ]
        hcat = parts[0] if heads == 1 else jnp.concatenate(parts, axis=1)
        h1 = jnp.maximum(hcat + b_ref[...], 0.0)
        xl_ref[...] = jnp.dot(h1, wl_ref[...], preferred_element_type=_F32)
        xr_ref[...] = jnp.dot(h1, wr_ref[...], preferred_element_type=_F32)

    return body


def _norm_proj(num, den, b, wl, wr, blk=1000):
    _, n, d = num.shape
    heads = den.shape[2]
    d2 = wl.shape[1]
    grid = (n + blk - 1) // blk
    return pl.pallas_call(
        _make_norm_body(heads, d // heads),
        grid=(grid,),
        in_specs=[
            pl.BlockSpec((_NC, blk, d), lambda i: (0, i, 0)),
            pl.BlockSpec((_NC, blk, heads), lambda i: (0, i, 0)),
            pl.BlockSpec((1, d), lambda i: (0, 0)),
            pl.BlockSpec((d, d2), lambda i: (0, 0)),
            pl.BlockSpec((d, d2), lambda i: (0, 0)),
        ],
        out_specs=[
            pl.BlockSpec((blk, d2), lambda i: (i, 0)),
            pl.BlockSpec((blk, d2), lambda i: (i, 0)),
        ],
        out_shape=[
            jax.ShapeDtypeStruct((n, d2), _F32),
            jax.ShapeDtypeStruct((n, d2), _F32),
        ],
    )(num, den, b.reshape(1, d), wl, wr)


def _make_final_body(num_graphs):
    def body(num_ref, den_ref, b2_ref, batch_ref, g1w_ref, g1b_ref, g2w_ref,
             g2b_ref, l1w_ref, l1b_ref, l2w_ref, l2b_ref, out_ref):
        n = num_ref[0] + num_ref[1]
        dsum = den_ref[0] + den_ref[1]
        h2 = jnp.maximum(n / (dsum + 1e-16) + b2_ref[...], 0.0)
        z1 = jnp.maximum(
            jnp.dot(h2, g1w_ref[...], preferred_element_type=_F32)
            + g1b_ref[...], 0.0)
        gate = jnp.dot(z1, g2w_ref[...], preferred_element_type=_F32) \
            + g2b_ref[...]
        gex = jnp.exp(gate)
        nn = h2.shape[0]
        seg = lax.broadcasted_iota(jnp.int32, (num_graphs, nn), 0)
        onehot = jnp.where(seg == batch_ref[...], 1.0, 0.0)
        pnum = jnp.dot(onehot, h2 * gex, preferred_element_type=_F32)
        gden = jnp.dot(onehot, gex, preferred_element_type=_F32)
        pooled = pnum / (gden + 1e-16)
        z = jnp.maximum(
            jnp.dot(pooled, l1w_ref[...], preferred_element_type=_F32)
            + l1b_ref[...], 0.0)
        out_ref[...] = jnp.dot(z, l2w_ref[...],
                               preferred_element_type=_F32) + l2b_ref[...]

    return body


def _final(num, den, b2, batch, g1w, g1b, g2w, g2b, l1w, l1b, l2w, l2b,
           num_graphs=64):
    _, n, d = num.shape
    return pl.pallas_call(
        _make_final_body(num_graphs),
        out_shape=jax.ShapeDtypeStruct((num_graphs, 1), _F32),
    )(num, den, b2.reshape(1, d), batch.reshape(1, n), g1w,
      g1b.reshape(1, d), g2w, g2b.reshape(1, 1), l1w, l1b.reshape(1, d),
      l2w, l2b.reshape(1, 1))


def kernel(x, edge_index, batch, Wl1, Wr1, att1, b1, Wl2, Wr2, att2, b2,
           g1W, g1b, g2W, g2b, l1W, l1b, l2W, l2b):
    src = edge_index[0]
    dst = edge_index[1]

    xl1, xr1 = _project(x, Wl1, Wr1)
    num1, den1 = _edge_sweep(xl1, xr1, src, dst, att1)
    xl2, xr2 = _norm_proj(num1, den1, b1, Wl2, Wr2)
    num2, den2 = _edge_sweep(xl2, xr2, src, dst, att2)
    out = _final(num2, den2, b2, batch, g1W, g1b, g2W, g2b, l1W, l1b,
                 l2W, l2b)
    return out.reshape(-1)


# speedup vs baseline: 29.3670x; 1.1833x over previous
import functools

import jax
import jax.numpy as jnp
from jax import lax
from jax.experimental import pallas as pl
from jax.experimental.pallas import tpu as pltpu
from jax.experimental.pallas import tpu_sc as plsc

_L = 16
_C = 128
_NC = 2
_NS = 16
_F32 = jnp.float32


def _proj_body(x_ref, wl_ref, wr_ref, xl_ref, xr_ref):
    xb = x_ref[...]
    xl_ref[...] = jnp.dot(xb, wl_ref[...], preferred_element_type=_F32)
    xr_ref[...] = jnp.dot(xb, wr_ref[...], preferred_element_type=_F32)


def _project(x, wl, wr, blk=1000):
    n, k = x.shape
    d = wl.shape[1]
    grid = (n + blk - 1) // blk
    return pl.pallas_call(
        _proj_body,
        grid=(grid,),
        in_specs=[
            pl.BlockSpec((blk, k), lambda i: (i, 0)),
            pl.BlockSpec((k, d), lambda i: (0, 0)),
            pl.BlockSpec((k, d), lambda i: (0, 0)),
        ],
        out_specs=[
            pl.BlockSpec((blk, d), lambda i: (i, 0)),
            pl.BlockSpec((blk, d), lambda i: (i, 0)),
        ],
        out_shape=[
            jax.ShapeDtypeStruct((n, d), _F32),
            jax.ShapeDtypeStruct((n, d), _F32),
        ],
    )(x, wl, wr)


def _make_edge_kernel(n_nodes, d, heads, n_edges):
    ch = d // heads
    assert n_edges % _C == 0
    n_chunks = n_edges // _C
    nw = _NC * _NS
    jmax = n_chunks // nw
    nleft = n_chunks - jmax * nw
    assert jmax % 2 == 0 and jmax >= 4
    assert n_nodes % _NS == 0
    rps = (n_nodes // _NS) & ~7
    tail = n_nodes - rps * _NS
    assert tail % 8 == 0

    mesh = plsc.VectorSubcoreMesh(core_axis_name="c", subcore_axis_name="s")

    @functools.partial(
        pl.kernel,
        out_type=(
            jax.ShapeDtypeStruct((_NC, n_nodes, d), _F32),
            jax.ShapeDtypeStruct((_NC, n_nodes, heads), _F32),
        ),
        mesh=mesh,
        compiler_params=pltpu.CompilerParams(needs_layout_passes=False,
                                             use_tc_tiling_on_sc=False),
        scratch_types=[
            pltpu.VMEM((2, _C), jnp.int32),
            pltpu.VMEM((2, _C), jnp.int32),
            pltpu.VMEM((_C,), jnp.int32),
            pltpu.VMEM((_C,), jnp.int32),
            pltpu.VMEM((_C, d), _F32),
            pltpu.VMEM((_C, d), _F32),
            pltpu.VMEM((_C, d), _F32),
            pltpu.VMEM((_C, d), _F32),
            pltpu.VMEM((_C, d), _F32),
            pltpu.VMEM((_C, d), _F32),
            pltpu.VMEM((_C, heads), _F32),
            pltpu.VMEM((_C, heads), _F32),
            pltpu.VMEM((d * _L,), _F32),
            pltpu.VMEM_SHARED((n_nodes, d), _F32),
            pltpu.VMEM_SHARED((n_nodes, heads), _F32),
            pltpu.SemaphoreType.DMA,
            pltpu.SemaphoreType.DMA,
            pltpu.SemaphoreType.DMA,
            pltpu.SemaphoreType.DMA,
            pltpu.SemaphoreType.DMA,
            pltpu.SemaphoreType.DMA,
        ],
    )
    def edge_kernel(edge_hbm, xl_hbm, xr_hbm, attb_hbm, znum_hbm,
                    zden_hbm, num_out, den_out, idx0, idx1, sidx0, sidx1,
                    xl0, xl1, xr0, xr1, sc0, sc1, pb0, pb1, att_v,
                    num_acc, den_acc, sem_i0, sem_i1, sem_g0, sem_g1,
                    sem_s0, sem_s1):
        cid = lax.axis_index("c")
        sid = lax.axis_index("s")
        wid = sid * _NC + cid
        iota = lax.iota(jnp.int32, _L)

        idx_v = (idx0, idx1)
        sidx = (sidx0, sidx1)
        xl_rows = (xl0, xl1)
        xr_rows = (xr0, xr1)
        scaled = (sc0, sc1)
        pbuf = (pb0, pb1)
        sem_i = (sem_i0, sem_i1)
        sem_g = (sem_g0, sem_g1)
        sem_s = (sem_s0, sem_s1)

        pltpu.sync_copy(attb_hbm, att_v)

        r0 = sid * rps
        pltpu.sync_copy(znum_hbm.at[pl.ds(0, rps)], num_acc.at[pl.ds(r0, rps)])
        pltpu.sync_copy(zden_hbm.at[pl.ds(0, rps)], den_acc.at[pl.ds(r0, rps)])
        if tail:
            @pl.when(sid == _NS - 1)
            def _():
                t0 = rps * _NS
                pltpu.sync_copy(znum_hbm.at[pl.ds(0, tail)],
                                num_acc.at[pl.ds(t0, tail)])
                pltpu.sync_copy(zden_hbm.at[pl.ds(0, tail)],
                                den_acc.at[pl.ds(t0, tail)])
        plsc.subcore_barrier()

        def chunk_base(j):
            return (wid + j * nw) * _C

        def issue_idx(s, j):
            return pltpu.async_copy(
                edge_hbm.at[:, pl.ds(chunk_base(j), _C)], idx_v[s], sem_i[s])

        def issue_gathers(s):
            g0 = pltpu.async_copy(xl_hbm.at[idx_v[s].at[0]], xl_rows[s],
                                  sem_g[s])
            g1 = pltpu.async_copy(xr_hbm.at[idx_v[s].at[1]], xr_rows[s],
                                  sem_g[s])
            return g0, g1

        def wait_gathers(s):
            pltpu.make_async_copy(xl_hbm.at[idx_v[s].at[0]], xl_rows[s],
                                  sem_g[s]).wait()
            pltpu.make_async_copy(xr_hbm.at[idx_v[s].at[1]], xr_rows[s],
                                  sem_g[s]).wait()

        def wait_idx(s, j):
            pltpu.make_async_copy(
                edge_hbm.at[:, pl.ds(chunk_base(j), _C)], idx_v[s],
                sem_i[s]).wait()

        def issue_scatters(s):
            pltpu.async_copy(scaled[s], num_acc.at[sidx[s]], sem_s[s],
                             add=True)
            pltpu.async_copy(pbuf[s], den_acc.at[sidx[s]], sem_s[s],
                             add=True)

        def wait_scatters(s):
            pltpu.make_async_copy(scaled[s], num_acc.at[sidx[s]],
                                  sem_s[s]).wait()
            pltpu.make_async_copy(pbuf[s], den_acc.at[sidx[s]],
                                  sem_s[s]).wait()

        def snapshot_dst(s):
            for t in range(_C // _L):
                sidx[s][pl.ds(t * _L, _L)] = idx_v[s][1, pl.ds(t * _L, _L)]

        def compute(s):
            for h in range(heads):
                hvec = jnp.full((_L,), h, jnp.int32)

                @pl.loop(0, _C // _L)
                def _(g):
                    rows = g * _L + iota
                    acc = jnp.zeros((_L,), _F32)
                    avals = []
                    for c in range(h * ch, (h + 1) * ch):
                        cvec = jnp.full((_L,), c, jnp.int32)
                        a = plsc.load_gather(xl_rows[s], [rows, cvec])
                        b = plsc.load_gather(xr_rows[s], [rows, cvec])
                        m = a + b
                        m = jnp.where(m >= 0.0, m, 0.2 * m)
                        att = att_v[pl.ds(c * _L, _L)]
                        acc = acc + m * att
                        avals.append(a)
                    p = jnp.exp(acc)
                    plsc.store_scatter(pbuf[s], [rows, hvec], p)
                    for c in range(h * ch, (h + 1) * ch):
                        cvec = jnp.full((_L,), c, jnp.int32)
                        plsc.store_scatter(scaled[s], [rows, cvec],
                                           avals[c - h * ch] * p)

        def slot_step(s, j, *, do_idx=True, do_next=True, do_waitsc=True):
            wait_gathers(s)
            if do_waitsc:
                wait_scatters(s)
            snapshot_dst(s)
            if do_idx:
                issue_idx(s, j + 2)
            if do_next:
                wait_idx(s ^ 1, j + 1)
                issue_gathers(s ^ 1)
            compute(s)
            issue_scatters(s)

        issue_idx(0, 0)
        wait_idx(0, 0)
        issue_gathers(0)
        issue_idx(1, 1)

        slot_step(0, 0, do_waitsc=False)
        slot_step(1, 1, do_waitsc=False)

        def pair_body(k, carry):
            j0 = 2 * k
            slot_step(0, j0)
            slot_step(1, j0 + 1)
            return carry

        lax.fori_loop(1, jmax // 2 - 1, pair_body, 0)

        slot_step(0, jmax - 2, do_idx=False)
        slot_step(1, jmax - 1, do_idx=False, do_next=False)
        wait_scatters(0)
        wait_scatters(1)

        if nleft:
            @pl.when(wid < nleft)
            def _():
                base = (jmax * nw + wid) * _C
                pltpu.sync_copy(edge_hbm.at[:, pl.ds(base, _C)], idx_v[0])
                g0, g1 = issue_gathers(0)
                g0.wait()
                g1.wait()
                snapshot_dst(0)
                compute(0)
                issue_scatters(0)
                wait_scatters(0)

        plsc.subcore_barrier()
        pltpu.sync_copy(num_acc.at[pl.ds(r0, rps)],
                        num_out.at[cid, pl.ds(r0, rps)])
        pltpu.sync_copy(den_acc.at[pl.ds(r0, rps)],
                        den_out.at[cid, pl.ds(r0, rps)])
        if tail:
            @pl.when(sid == _NS - 1)
            def _():
                t0 = rps * _NS
                pltpu.sync_copy(num_acc.at[pl.ds(t0, tail)],
                                num_out.at[cid, pl.ds(t0, tail)])
                pltpu.sync_copy(den_acc.at[pl.ds(t0, tail)],
                                den_out.at[cid, pl.ds(t0, tail)])

    return edge_kernel


def _edge_sweep(xl, xr, edge_index, att):
    n, d = xl.shape
    heads = att.shape[0]
    e = edge_index.shape[1]
    attb = jnp.broadcast_to(att.reshape(d, 1), (d, _L)).reshape(d * _L)
    znum = jnp.zeros((n // _NS, d), _F32)
    zden = jnp.zeros((n // _NS, heads), _F32)
    k = _make_edge_kernel(n, d, heads, e)
    num, den = k(edge_index, xl, xr, attb, znum, zden)
    return num, den


def _make_norm_body(heads, ch):
    def body(num_ref, den_ref, b_ref, wl_ref, wr_ref, xl_ref, xr_ref):
        n = num_ref[0] + num_ref[1]
        dsum = den_ref[0] + den_ref[1]
        parts = [
            n[:, h * ch:(h + 1) * ch] / (dsum[:, h:h + 1] + 1e-16)
            for h in range(heads)
        ]
        hcat = parts[0] if heads == 1 else jnp.concatenate(parts, axis=1)
        h1 = jnp.maximum(hcat + b_ref[...], 0.0)
        xl_ref[...] = jnp.dot(h1, wl_ref[...], preferred_element_type=_F32)
        xr_ref[...] = jnp.dot(h1, wr_ref[...], preferred_element_type=_F32)

    return body


def _norm_proj(num, den, b, wl, wr, blk=1000):
    _, n, d = num.shape
    heads = den.shape[2]
    d2 = wl.shape[1]
    grid = (n + blk - 1) // blk
    return pl.pallas_call(
        _make_norm_body(heads, d // heads),
        grid=(grid,),
        in_specs=[
            pl.BlockSpec((_NC, blk, d), lambda i: (0, i, 0)),
            pl.BlockSpec((_NC, blk, heads), lambda i: (0, i, 0)),
            pl.BlockSpec((1, d), lambda i: (0, 0)),
            pl.BlockSpec((d, d2), lambda i: (0, 0)),
            pl.BlockSpec((d, d2), lambda i: (0, 0)),
        ],
        out_specs=[
            pl.BlockSpec((blk, d2), lambda i: (i, 0)),
            pl.BlockSpec((blk, d2), lambda i: (i, 0)),
        ],
        out_shape=[
            jax.ShapeDtypeStruct((n, d2), _F32),
            jax.ShapeDtypeStruct((n, d2), _F32),
        ],
    )(num, den, b.reshape(1, d), wl, wr)


def _make_final_body(num_graphs):
    def body(num_ref, den_ref, b2_ref, batch_ref, g1w_ref, g1b_ref, g2w_ref,
             g2b_ref, l1w_ref, l1b_ref, l2w_ref, l2b_ref, out_ref):
        n = num_ref[0] + num_ref[1]
        dsum = den_ref[0] + den_ref[1]
        h2 = jnp.maximum(n / (dsum + 1e-16) + b2_ref[...], 0.0)
        z1 = jnp.maximum(
            jnp.dot(h2, g1w_ref[...], preferred_element_type=_F32)
            + g1b_ref[...], 0.0)
        gate = jnp.dot(z1, g2w_ref[...], preferred_element_type=_F32) \
            + g2b_ref[...]
        gex = jnp.exp(gate)
        nn = h2.shape[0]
        seg = lax.broadcasted_iota(jnp.int32, (num_graphs, nn), 0)
        onehot = jnp.where(seg == batch_ref[...], 1.0, 0.0)
        pnum = jnp.dot(onehot, h2 * gex, preferred_element_type=_F32)
        gden = jnp.dot(onehot, gex, preferred_element_type=_F32)
        pooled = pnum / (gden + 1e-16)
        z = jnp.maximum(
            jnp.dot(pooled, l1w_ref[...], preferred_element_type=_F32)
            + l1b_ref[...], 0.0)
        out_ref[...] = jnp.dot(z, l2w_ref[...],
                               preferred_element_type=_F32) + l2b_ref[...]

    return body


def _final(num, den, b2, batch, g1w, g1b, g2w, g2b, l1w, l1b, l2w, l2b,
           num_graphs=64):
    _, n, d = num.shape
    return pl.pallas_call(
        _make_final_body(num_graphs),
        out_shape=jax.ShapeDtypeStruct((num_graphs, 1), _F32),
    )(num, den, b2.reshape(1, d), batch.reshape(1, n), g1w,
      g1b.reshape(1, d), g2w, g2b.reshape(1, 1), l1w, l1b.reshape(1, d),
      l2w, l2b.reshape(1, 1))


def kernel(x, edge_index, batch, Wl1, Wr1, att1, b1, Wl2, Wr2, att2, b2,
           g1W, g1b, g2W, g2b, l1W, l1b, l2W, l2b):
    xl1, xr1 = _project(x, Wl1, Wr1)
    num1, den1 = _edge_sweep(xl1, xr1, edge_index, att1)
    xl2, xr2 = _norm_proj(num1, den1, b1, Wl2, Wr2)
    num2, den2 = _edge_sweep(xl2, xr2, edge_index, att2)
    out = _final(num2, den2, b2, batch, g1W, g1b, g2W, g2b, l1W, l1b,
                 l2W, l2b)
    return out.reshape(-1)


# speedup vs baseline: 48.6740x; 1.6574x over previous
import functools

import jax
import jax.numpy as jnp
from jax import lax
from jax.experimental import pallas as pl
from jax.experimental.pallas import tpu as pltpu
from jax.experimental.pallas import tpu_sc as plsc

_L = 16
_C = 128
_NC = 2
_NS = 16
_F32 = jnp.float32


def _proj_body(x_ref, wl_ref, wr_ref, xl_ref, xr_ref):
    xb = x_ref[...]
    xl_ref[...] = jnp.dot(xb, wl_ref[...], preferred_element_type=_F32)
    xr_ref[...] = jnp.dot(xb, wr_ref[...], preferred_element_type=_F32)


def _project(x, wl, wr, blk=1000):
    n, k = x.shape
    d = wl.shape[1]
    grid = (n + blk - 1) // blk
    return pl.pallas_call(
        _proj_body,
        grid=(grid,),
        in_specs=[
            pl.BlockSpec((blk, k), lambda i: (i, 0)),
            pl.BlockSpec((k, d), lambda i: (0, 0)),
            pl.BlockSpec((k, d), lambda i: (0, 0)),
        ],
        out_specs=[
            pl.BlockSpec((blk, d), lambda i: (i, 0)),
            pl.BlockSpec((blk, d), lambda i: (i, 0)),
        ],
        out_shape=[
            jax.ShapeDtypeStruct((n, d), _F32),
            jax.ShapeDtypeStruct((n, d), _F32),
        ],
    )(x, wl, wr)


def _make_edge_kernel(n_nodes, d, heads, n_edges):
    ch = d // heads
    assert n_edges % _C == 0
    n_chunks = n_edges // _C
    nw = _NC * _NS
    jmax = n_chunks // nw
    nleft = n_chunks - jmax * nw
    assert jmax % 2 == 0 and jmax >= 4
    assert n_nodes % _NS == 0
    rps = (n_nodes // _NS) & ~7
    tail = n_nodes - rps * _NS
    assert tail % 8 == 0

    mesh = plsc.VectorSubcoreMesh(core_axis_name="c", subcore_axis_name="s")

    @functools.partial(
        pl.kernel,
        out_type=(
            jax.ShapeDtypeStruct((_NC, n_nodes, d), _F32),
            jax.ShapeDtypeStruct((_NC, n_nodes, heads), _F32),
        ),
        mesh=mesh,
        compiler_params=pltpu.CompilerParams(needs_layout_passes=False,
                                             use_tc_tiling_on_sc=False),
        scratch_types=[
            pltpu.VMEM((2, _C), jnp.int32),
            pltpu.VMEM((2, _C), jnp.int32),
            pltpu.VMEM((_C,), jnp.int32),
            pltpu.VMEM((_C,), jnp.int32),
            pltpu.VMEM((_C, d), _F32),
            pltpu.VMEM((_C, d), _F32),
            pltpu.VMEM((_C, d), _F32),
            pltpu.VMEM((_C, d), _F32),
            pltpu.VMEM((_C, d), _F32),
            pltpu.VMEM((_C, d), _F32),
            pltpu.VMEM((_C, heads), _F32),
            pltpu.VMEM((_C, heads), _F32),
            pltpu.VMEM((d,), _F32),
            pltpu.VMEM_SHARED((n_nodes, d), _F32),
            pltpu.VMEM_SHARED((n_nodes, heads), _F32),
            pltpu.SemaphoreType.DMA,
            pltpu.SemaphoreType.DMA,
            pltpu.SemaphoreType.DMA,
            pltpu.SemaphoreType.DMA,
            pltpu.SemaphoreType.DMA,
            pltpu.SemaphoreType.DMA,
        ],
    )
    def edge_kernel(edge_hbm, xl_hbm, xr_hbm, attb_hbm, znum_hbm,
                    zden_hbm, num_out, den_out, idx0, idx1, sidx0, sidx1,
                    xl0, xl1, xr0, xr1, sc0, sc1, pb0, pb1, att_v,
                    num_acc, den_acc, sem_i0, sem_i1, sem_g0, sem_g1,
                    sem_s0, sem_s1):
        cid = lax.axis_index("c")
        sid = lax.axis_index("s")
        wid = sid * _NC + cid
        iota = lax.iota(jnp.int32, _L)

        idx_v = (idx0, idx1)
        sidx = (sidx0, sidx1)
        xl_rows = (xl0, xl1)
        xr_rows = (xr0, xr1)
        scaled = (sc0, sc1)
        pbuf = (pb0, pb1)
        sem_i = (sem_i0, sem_i1)
        sem_g = (sem_g0, sem_g1)
        sem_s = (sem_s0, sem_s1)

        pltpu.sync_copy(attb_hbm, att_v)

        r0 = sid * rps
        pltpu.sync_copy(znum_hbm.at[pl.ds(0, rps)], num_acc.at[pl.ds(r0, rps)])
        pltpu.sync_copy(zden_hbm.at[pl.ds(0, rps)], den_acc.at[pl.ds(r0, rps)])
        if tail:
            @pl.when(sid == _NS - 1)
            def _():
                t0 = rps * _NS
                pltpu.sync_copy(znum_hbm.at[pl.ds(0, tail)],
                                num_acc.at[pl.ds(t0, tail)])
                pltpu.sync_copy(zden_hbm.at[pl.ds(0, tail)],
                                den_acc.at[pl.ds(t0, tail)])
        plsc.subcore_barrier()

        def chunk_base(j):
            return (wid + j * nw) * _C

        def issue_idx(s, j):
            return pltpu.async_copy(
                edge_hbm.at[:, pl.ds(chunk_base(j), _C)], idx_v[s], sem_i[s])

        def issue_gathers(s):
            g0 = pltpu.async_copy(xl_hbm.at[idx_v[s].at[0]], xl_rows[s],
                                  sem_g[s])
            g1 = pltpu.async_copy(xr_hbm.at[idx_v[s].at[1]], xr_rows[s],
                                  sem_g[s])
            return g0, g1

        def wait_gathers(s):
            pltpu.make_async_copy(xl_hbm.at[idx_v[s].at[0]], xl_rows[s],
                                  sem_g[s]).wait()
            pltpu.make_async_copy(xr_hbm.at[idx_v[s].at[1]], xr_rows[s],
                                  sem_g[s]).wait()

        def wait_idx(s, j):
            pltpu.make_async_copy(
                edge_hbm.at[:, pl.ds(chunk_base(j), _C)], idx_v[s],
                sem_i[s]).wait()

        def issue_scatters(s):
            pltpu.async_copy(scaled[s], num_acc.at[sidx[s]], sem_s[s],
                             add=True)
            pltpu.async_copy(pbuf[s], den_acc.at[sidx[s]], sem_s[s],
                             add=True)

        def wait_scatters(s):
            pltpu.make_async_copy(scaled[s], num_acc.at[sidx[s]],
                                  sem_s[s]).wait()
            pltpu.make_async_copy(pbuf[s], den_acc.at[sidx[s]],
                                  sem_s[s]).wait()

        def snapshot_dst(s):
            for t in range(_C // _L):
                sidx[s][pl.ds(t * _L, _L)] = idx_v[s][1, pl.ds(t * _L, _L)]

        nseg = d // _L
        sph = nseg // heads
        mask0 = iota == 0
        unroll = 8

        def compute(s):
            attv = [att_v[pl.ds(q * _L, _L)] for q in range(nseg)]

            @pl.loop(0, _C // unroll)
            def _(eg):
                for k in range(unroll):
                    e = eg * unroll + k
                    th = [None] * heads
                    for q in range(nseg):
                        a = xl_rows[s][e, pl.ds(q * _L, _L)]
                        b = xr_rows[s][e, pl.ds(q * _L, _L)]
                        m = a + b
                        m = jnp.where(m >= 0.0, m, 0.2 * m)
                        t = m * attv[q]
                        h = q // sph
                        th[h] = t if th[h] is None else th[h] + t
                    evec = jnp.full((_L,), e, jnp.int32)
                    for h in range(heads):
                        eh = jnp.sum(th[h])
                        pv = jnp.exp(jnp.full((_L,), eh, _F32))
                        plsc.store_scatter(
                            pbuf[s], [evec, jnp.full((_L,), h, jnp.int32)],
                            pv, mask=mask0)
                        for q in range(h * sph, (h + 1) * sph):
                            a = xl_rows[s][e, pl.ds(q * _L, _L)]
                            scaled[s][e, pl.ds(q * _L, _L)] = a * pv

        def slot_step(s, j, *, do_idx=True, do_next=True, do_waitsc=True):
            wait_gathers(s)
            if do_waitsc:
                wait_scatters(s)
            snapshot_dst(s)
            if do_idx:
                issue_idx(s, j + 2)
            if do_next:
                wait_idx(s ^ 1, j + 1)
                issue_gathers(s ^ 1)
            compute(s)
            issue_scatters(s)

        issue_idx(0, 0)
        wait_idx(0, 0)
        issue_gathers(0)
        issue_idx(1, 1)

        slot_step(0, 0, do_waitsc=False)
        slot_step(1, 1, do_waitsc=False)

        def pair_body(k, carry):
            j0 = 2 * k
            slot_step(0, j0)
            slot_step(1, j0 + 1)
            return carry

        lax.fori_loop(1, jmax // 2 - 1, pair_body, 0)

        slot_step(0, jmax - 2, do_idx=False)
        slot_step(1, jmax - 1, do_idx=False, do_next=False)
        wait_scatters(0)
        wait_scatters(1)

        if nleft:
            @pl.when(wid < nleft)
            def _():
                base = (jmax * nw + wid) * _C
                pltpu.sync_copy(edge_hbm.at[:, pl.ds(base, _C)], idx_v[0])
                g0, g1 = issue_gathers(0)
                g0.wait()
                g1.wait()
                snapshot_dst(0)
                compute(0)
                issue_scatters(0)
                wait_scatters(0)

        plsc.subcore_barrier()
        pltpu.sync_copy(num_acc.at[pl.ds(r0, rps)],
                        num_out.at[cid, pl.ds(r0, rps)])
        pltpu.sync_copy(den_acc.at[pl.ds(r0, rps)],
                        den_out.at[cid, pl.ds(r0, rps)])
        if tail:
            @pl.when(sid == _NS - 1)
            def _():
                t0 = rps * _NS
                pltpu.sync_copy(num_acc.at[pl.ds(t0, tail)],
                                num_out.at[cid, pl.ds(t0, tail)])
                pltpu.sync_copy(den_acc.at[pl.ds(t0, tail)],
                                den_out.at[cid, pl.ds(t0, tail)])

    return edge_kernel


def _edge_sweep(xl, xr, edge_index, att):
    n, d = xl.shape
    heads = att.shape[0]
    e = edge_index.shape[1]
    attb = att.reshape(d)
    znum = jnp.zeros((n // _NS, d), _F32)
    zden = jnp.zeros((n // _NS, heads), _F32)
    k = _make_edge_kernel(n, d, heads, e)
    num, den = k(edge_index, xl, xr, attb, znum, zden)
    return num, den


def _make_norm_body(heads, ch):
    def body(num_ref, den_ref, b_ref, wl_ref, wr_ref, xl_ref, xr_ref):
        n = num_ref[0] + num_ref[1]
        dsum = den_ref[0] + den_ref[1]
        parts = [
            n[:, h * ch:(h + 1) * ch] / (dsum[:, h:h + 1] + 1e-16)
            for h in range(heads)
        ]
        hcat = parts[0] if heads == 1 else jnp.concatenate(parts, axis=1)
        h1 = jnp.maximum(hcat + b_ref[...], 0.0)
        xl_ref[...] = jnp.dot(h1, wl_ref[...], preferred_element_type=_F32)
        xr_ref[...] = jnp.dot(h1, wr_ref[...], preferred_element_type=_F32)

    return body


def _norm_proj(num, den, b, wl, wr, blk=1000):
    _, n, d = num.shape
    heads = den.shape[2]
    d2 = wl.shape[1]
    grid = (n + blk - 1) // blk
    return pl.pallas_call(
        _make_norm_body(heads, d // heads),
        grid=(grid,),
        in_specs=[
            pl.BlockSpec((_NC, blk, d), lambda i: (0, i, 0)),
            pl.BlockSpec((_NC, blk, heads), lambda i: (0, i, 0)),
            pl.BlockSpec((1, d), lambda i: (0, 0)),
            pl.BlockSpec((d, d2), lambda i: (0, 0)),
            pl.BlockSpec((d, d2), lambda i: (0, 0)),
        ],
        out_specs=[
            pl.BlockSpec((blk, d2), lambda i: (i, 0)),
            pl.BlockSpec((blk, d2), lambda i: (i, 0)),
        ],
        out_shape=[
            jax.ShapeDtypeStruct((n, d2), _F32),
            jax.ShapeDtypeStruct((n, d2), _F32),
        ],
    )(num, den, b.reshape(1, d), wl, wr)


def _make_final_body(num_graphs):
    def body(num_ref, den_ref, b2_ref, batch_ref, g1w_ref, g1b_ref, g2w_ref,
             g2b_ref, l1w_ref, l1b_ref, l2w_ref, l2b_ref, out_ref):
        n = num_ref[0] + num_ref[1]
        dsum = den_ref[0] + den_ref[1]
        h2 = jnp.maximum(n / (dsum + 1e-16) + b2_ref[...], 0.0)
        z1 = jnp.maximum(
            jnp.dot(h2, g1w_ref[...], preferred_element_type=_F32)
            + g1b_ref[...], 0.0)
        gate = jnp.dot(z1, g2w_ref[...], preferred_element_type=_F32) \
            + g2b_ref[...]
        gex = jnp.exp(gate)
        nn = h2.shape[0]
        seg = lax.broadcasted_iota(jnp.int32, (num_graphs, nn), 0)
        onehot = jnp.where(seg == batch_ref[...], 1.0, 0.0)
        pnum = jnp.dot(onehot, h2 * gex, preferred_element_type=_F32)
        gden = jnp.dot(onehot, gex, preferred_element_type=_F32)
        pooled = pnum / (gden + 1e-16)
        z = jnp.maximum(
            jnp.dot(pooled, l1w_ref[...], preferred_element_type=_F32)
            + l1b_ref[...], 0.0)
        out_ref[...] = jnp.dot(z, l2w_ref[...],
                               preferred_element_type=_F32) + l2b_ref[...]

    return body


def _final(num, den, b2, batch, g1w, g1b, g2w, g2b, l1w, l1b, l2w, l2b,
           num_graphs=64):
    _, n, d = num.shape
    return pl.pallas_call(
        _make_final_body(num_graphs),
        out_shape=jax.ShapeDtypeStruct((num_graphs, 1), _F32),
    )(num, den, b2.reshape(1, d), batch.reshape(1, n), g1w,
      g1b.reshape(1, d), g2w, g2b.reshape(1, 1), l1w, l1b.reshape(1, d),
      l2w, l2b.reshape(1, 1))


def kernel(x, edge_index, batch, Wl1, Wr1, att1, b1, Wl2, Wr2, att2, b2,
           g1W, g1b, g2W, g2b, l1W, l1b, l2W, l2b):
    xl1, xr1 = _project(x, Wl1, Wr1)
    num1, den1 = _edge_sweep(xl1, xr1, edge_index, att1)
    xl2, xr2 = _norm_proj(num1, den1, b1, Wl2, Wr2)
    num2, den2 = _edge_sweep(xl2, xr2, edge_index, att2)
    out = _final(num2, den2, b2, batch, g1W, g1b, g2W, g2b, l1W, l1b,
                 l2W, l2b)
    return out.reshape(-1)


# speedup vs baseline: 136.1628x; 2.7974x over previous
import functools

import jax
import jax.numpy as jnp
from jax import lax
from jax.experimental import pallas as pl
from jax.experimental.pallas import tpu as pltpu
from jax.experimental.pallas import tpu_sc as plsc

_L = 16
_C = 128
_NC = 2
_NS = 16
_F32 = jnp.float32


def _proj_body(x_ref, wl_ref, wr_ref, xl_ref, xr_ref):
    xb = x_ref[...]
    xl_ref[...] = jnp.dot(xb, wl_ref[...], preferred_element_type=_F32)
    xr_ref[...] = jnp.dot(xb, wr_ref[...], preferred_element_type=_F32)


def _project(x, wl, wr, blk=1000):
    n, k = x.shape
    d = wl.shape[1]
    grid = (n + blk - 1) // blk
    return pl.pallas_call(
        _proj_body,
        grid=(grid,),
        in_specs=[
            pl.BlockSpec((blk, k), lambda i: (i, 0)),
            pl.BlockSpec((k, d), lambda i: (0, 0)),
            pl.BlockSpec((k, d), lambda i: (0, 0)),
        ],
        out_specs=[
            pl.BlockSpec((blk, d), lambda i: (i, 0)),
            pl.BlockSpec((blk, d), lambda i: (i, 0)),
        ],
        out_shape=[
            jax.ShapeDtypeStruct((n, d), _F32),
            jax.ShapeDtypeStruct((n, d), _F32),
        ],
    )(x, wl, wr)


def _make_edge_kernel(n_nodes, d, heads, n_edges):
    ch = d // heads
    assert n_edges % _C == 0
    n_chunks = n_edges // _C
    nw = _NC * _NS
    jmax = n_chunks // nw
    nleft = n_chunks - jmax * nw
    assert jmax % 2 == 0 and jmax >= 4
    assert n_nodes % _NS == 0
    rps = (n_nodes // _NS) & ~7
    tail = n_nodes - rps * _NS
    assert tail % 8 == 0

    mesh = plsc.VectorSubcoreMesh(core_axis_name="c", subcore_axis_name="s")

    @functools.partial(
        pl.kernel,
        out_type=(
            jax.ShapeDtypeStruct((_NC, n_nodes, d), _F32),
            jax.ShapeDtypeStruct((_NC, n_nodes, heads), _F32),
        ),
        mesh=mesh,
        compiler_params=pltpu.CompilerParams(needs_layout_passes=False,
                                             use_tc_tiling_on_sc=False),
        scratch_types=[
            pltpu.VMEM((2, _C), jnp.int32),
            pltpu.VMEM((2, _C), jnp.int32),
            pltpu.VMEM((_C,), jnp.int32),
            pltpu.VMEM((_C,), jnp.int32),
            pltpu.VMEM((_C, d), _F32),
            pltpu.VMEM((_C, d), _F32),
            pltpu.VMEM((_C, d), _F32),
            pltpu.VMEM((_C, d), _F32),
            pltpu.VMEM((_C, d), _F32),
            pltpu.VMEM((_C, d), _F32),
            pltpu.VMEM((_C, heads), _F32),
            pltpu.VMEM((_C, heads), _F32),
            pltpu.VMEM((d,), _F32),
            pltpu.VMEM_SHARED((n_nodes, d), _F32),
            pltpu.VMEM_SHARED((n_nodes, heads), _F32),
            pltpu.SemaphoreType.DMA,
            pltpu.SemaphoreType.DMA,
            pltpu.SemaphoreType.DMA,
            pltpu.SemaphoreType.DMA,
            pltpu.SemaphoreType.DMA,
            pltpu.SemaphoreType.DMA,
        ],
    )
    def edge_kernel(edge_hbm, xl_hbm, xr_hbm, attb_hbm, znum_hbm,
                    zden_hbm, num_out, den_out, idx0, idx1, sidx0, sidx1,
                    xl0, xl1, xr0, xr1, sc0, sc1, pb0, pb1, att_v,
                    num_acc, den_acc, sem_i0, sem_i1, sem_g0, sem_g1,
                    sem_s0, sem_s1):
        cid = lax.axis_index("c")
        sid = lax.axis_index("s")
        wid = sid * _NC + cid
        iota = lax.iota(jnp.int32, _L)

        idx_v = (idx0, idx1)
        sidx = (sidx0, sidx1)
        xl_rows = (xl0, xl1)
        xr_rows = (xr0, xr1)
        scaled = (sc0, sc1)
        pbuf = (pb0, pb1)
        sem_i = (sem_i0, sem_i1)
        sem_g = (sem_g0, sem_g1)
        sem_s = (sem_s0, sem_s1)

        pltpu.sync_copy(attb_hbm, att_v)

        r0 = sid * rps
        pltpu.sync_copy(znum_hbm.at[pl.ds(0, rps)], num_acc.at[pl.ds(r0, rps)])
        pltpu.sync_copy(zden_hbm.at[pl.ds(0, rps)], den_acc.at[pl.ds(r0, rps)])
        if tail:
            @pl.when(sid == _NS - 1)
            def _():
                t0 = rps * _NS
                pltpu.sync_copy(znum_hbm.at[pl.ds(0, tail)],
                                num_acc.at[pl.ds(t0, tail)])
                pltpu.sync_copy(zden_hbm.at[pl.ds(0, tail)],
                                den_acc.at[pl.ds(t0, tail)])
        plsc.subcore_barrier()

        def chunk_base(j):
            return (wid + j * nw) * _C

        def issue_idx(s, j):
            return pltpu.async_copy(
                edge_hbm.at[:, pl.ds(chunk_base(j), _C)], idx_v[s], sem_i[s])

        def issue_gathers(s):
            g0 = pltpu.async_copy(xl_hbm.at[idx_v[s].at[0]], xl_rows[s],
                                  sem_g[s])
            g1 = pltpu.async_copy(xr_hbm.at[idx_v[s].at[1]], xr_rows[s],
                                  sem_g[s])
            return g0, g1

        def wait_gathers(s):
            pltpu.make_async_copy(xl_hbm.at[idx_v[s].at[0]], xl_rows[s],
                                  sem_g[s]).wait()
            pltpu.make_async_copy(xr_hbm.at[idx_v[s].at[1]], xr_rows[s],
                                  sem_g[s]).wait()

        def wait_idx(s, j):
            pltpu.make_async_copy(
                edge_hbm.at[:, pl.ds(chunk_base(j), _C)], idx_v[s],
                sem_i[s]).wait()

        def issue_scatters(s):
            pltpu.async_copy(scaled[s], num_acc.at[sidx[s]], sem_s[s],
                             add=True)
            pltpu.async_copy(pbuf[s], den_acc.at[sidx[s]], sem_s[s],
                             add=True)

        def wait_scatters(s):
            pltpu.make_async_copy(scaled[s], num_acc.at[sidx[s]],
                                  sem_s[s]).wait()
            pltpu.make_async_copy(pbuf[s], den_acc.at[sidx[s]],
                                  sem_s[s]).wait()

        def snapshot_dst(s):
            for t in range(_C // _L):
                sidx[s][pl.ds(t * _L, _L)] = idx_v[s][1, pl.ds(t * _L, _L)]

        nseg = d // _L
        sph = nseg // heads
        mask0 = iota == 0
        unroll = 4

        def compute(s):
            attv = [att_v[pl.ds(q * _L, _L)] for q in range(nseg)]

            @plsc.parallel_loop(0, _C // unroll)
            def _(eg):
                for k in range(unroll):
                    e = eg * unroll + k
                    th = [None] * heads
                    for q in range(nseg):
                        a = xl_rows[s][e, pl.ds(q * _L, _L)]
                        b = xr_rows[s][e, pl.ds(q * _L, _L)]
                        m = a + b
                        m = jnp.where(m >= 0.0, m, 0.2 * m)
                        t = m * attv[q]
                        h = q // sph
                        th[h] = t if th[h] is None else th[h] + t
                    evec = jnp.full((_L,), e, jnp.int32)
                    for h in range(heads):
                        eh = jnp.sum(th[h])
                        pv = jnp.exp(jnp.full((_L,), eh, _F32))
                        plsc.store_scatter(
                            pbuf[s], [evec, jnp.full((_L,), h, jnp.int32)],
                            pv, mask=mask0)
                        for q in range(h * sph, (h + 1) * sph):
                            a = xl_rows[s][e, pl.ds(q * _L, _L)]
                            scaled[s][e, pl.ds(q * _L, _L)] = a * pv

        def slot_step(s, j, *, do_idx=True, do_next=True, do_waitsc=True):
            wait_gathers(s)
            if do_waitsc:
                wait_scatters(s)
            snapshot_dst(s)
            if do_idx:
                issue_idx(s, j + 2)
            if do_next:
                wait_idx(s ^ 1, j + 1)
                issue_gathers(s ^ 1)
            compute(s)
            issue_scatters(s)

        issue_idx(0, 0)
        wait_idx(0, 0)
        issue_gathers(0)
        issue_idx(1, 1)

        slot_step(0, 0, do_waitsc=False)
        slot_step(1, 1, do_waitsc=False)

        def pair_body(k, carry):
            j0 = 2 * k
            slot_step(0, j0)
            slot_step(1, j0 + 1)
            return carry

        lax.fori_loop(1, jmax // 2 - 1, pair_body, 0)

        slot_step(0, jmax - 2, do_idx=False)
        slot_step(1, jmax - 1, do_idx=False, do_next=False)
        wait_scatters(0)
        wait_scatters(1)

        if nleft:
            @pl.when(wid < nleft)
            def _():
                base = (jmax * nw + wid) * _C
                pltpu.sync_copy(edge_hbm.at[:, pl.ds(base, _C)], idx_v[0])
                g0, g1 = issue_gathers(0)
                g0.wait()
                g1.wait()
                snapshot_dst(0)
                compute(0)
                issue_scatters(0)
                wait_scatters(0)

        plsc.subcore_barrier()
        pltpu.sync_copy(num_acc.at[pl.ds(r0, rps)],
                        num_out.at[cid, pl.ds(r0, rps)])
        pltpu.sync_copy(den_acc.at[pl.ds(r0, rps)],
                        den_out.at[cid, pl.ds(r0, rps)])
        if tail:
            @pl.when(sid == _NS - 1)
            def _():
                t0 = rps * _NS
                pltpu.sync_copy(num_acc.at[pl.ds(t0, tail)],
                                num_out.at[cid, pl.ds(t0, tail)])
                pltpu.sync_copy(den_acc.at[pl.ds(t0, tail)],
                                den_out.at[cid, pl.ds(t0, tail)])

    return edge_kernel


def _edge_sweep(xl, xr, edge_index, att):
    n, d = xl.shape
    heads = att.shape[0]
    e = edge_index.shape[1]
    attb = att.reshape(d)
    znum = jnp.zeros((n // _NS, d), _F32)
    zden = jnp.zeros((n // _NS, heads), _F32)
    k = _make_edge_kernel(n, d, heads, e)
    num, den = k(edge_index, xl, xr, attb, znum, zden)
    return num, den


def _make_norm_body(heads, ch):
    def body(num_ref, den_ref, b_ref, wl_ref, wr_ref, xl_ref, xr_ref):
        n = num_ref[0] + num_ref[1]
        dsum = den_ref[0] + den_ref[1]
        parts = [
            n[:, h * ch:(h + 1) * ch] / (dsum[:, h:h + 1] + 1e-16)
            for h in range(heads)
        ]
        hcat = parts[0] if heads == 1 else jnp.concatenate(parts, axis=1)
        h1 = jnp.maximum(hcat + b_ref[...], 0.0)
        xl_ref[...] = jnp.dot(h1, wl_ref[...], preferred_element_type=_F32)
        xr_ref[...] = jnp.dot(h1, wr_ref[...], preferred_element_type=_F32)

    return body


def _norm_proj(num, den, b, wl, wr, blk=1000):
    _, n, d = num.shape
    heads = den.shape[2]
    d2 = wl.shape[1]
    grid = (n + blk - 1) // blk
    return pl.pallas_call(
        _make_norm_body(heads, d // heads),
        grid=(grid,),
        in_specs=[
            pl.BlockSpec((_NC, blk, d), lambda i: (0, i, 0)),
            pl.BlockSpec((_NC, blk, heads), lambda i: (0, i, 0)),
            pl.BlockSpec((1, d), lambda i: (0, 0)),
            pl.BlockSpec((d, d2), lambda i: (0, 0)),
            pl.BlockSpec((d, d2), lambda i: (0, 0)),
        ],
        out_specs=[
            pl.BlockSpec((blk, d2), lambda i: (i, 0)),
            pl.BlockSpec((blk, d2), lambda i: (i, 0)),
        ],
        out_shape=[
            jax.ShapeDtypeStruct((n, d2), _F32),
            jax.ShapeDtypeStruct((n, d2), _F32),
        ],
    )(num, den, b.reshape(1, d), wl, wr)


def _make_final_body(num_graphs):
    def body(num_ref, den_ref, b2_ref, batch_ref, g1w_ref, g1b_ref, g2w_ref,
             g2b_ref, l1w_ref, l1b_ref, l2w_ref, l2b_ref, out_ref):
        n = num_ref[0] + num_ref[1]
        dsum = den_ref[0] + den_ref[1]
        h2 = jnp.maximum(n / (dsum + 1e-16) + b2_ref[...], 0.0)
        z1 = jnp.maximum(
            jnp.dot(h2, g1w_ref[...], preferred_element_type=_F32)
            + g1b_ref[...], 0.0)
        gate = jnp.dot(z1, g2w_ref[...], preferred_element_type=_F32) \
            + g2b_ref[...]
        gex = jnp.exp(gate)
        nn = h2.shape[0]
        seg = lax.broadcasted_iota(jnp.int32, (num_graphs, nn), 0)
        onehot = jnp.where(seg == batch_ref[...], 1.0, 0.0)
        pnum = jnp.dot(onehot, h2 * gex, preferred_element_type=_F32)
        gden = jnp.dot(onehot, gex, preferred_element_type=_F32)
        pooled = pnum / (gden + 1e-16)
        z = jnp.maximum(
            jnp.dot(pooled, l1w_ref[...], preferred_element_type=_F32)
            + l1b_ref[...], 0.0)
        out_ref[...] = jnp.dot(z, l2w_ref[...],
                               preferred_element_type=_F32) + l2b_ref[...]

    return body


def _final(num, den, b2, batch, g1w, g1b, g2w, g2b, l1w, l1b, l2w, l2b,
           num_graphs=64):
    _, n, d = num.shape
    return pl.pallas_call(
        _make_final_body(num_graphs),
        out_shape=jax.ShapeDtypeStruct((num_graphs, 1), _F32),
    )(num, den, b2.reshape(1, d), batch.reshape(1, n), g1w,
      g1b.reshape(1, d), g2w, g2b.reshape(1, 1), l1w, l1b.reshape(1, d),
      l2w, l2b.reshape(1, 1))


def kernel(x, edge_index, batch, Wl1, Wr1, att1, b1, Wl2, Wr2, att2, b2,
           g1W, g1b, g2W, g2b, l1W, l1b, l2W, l2b):
    xl1, xr1 = _project(x, Wl1, Wr1)
    num1, den1 = _edge_sweep(xl1, xr1, edge_index, att1)
    xl2, xr2 = _norm_proj(num1, den1, b1, Wl2, Wr2)
    num2, den2 = _edge_sweep(xl2, xr2, edge_index, att2)
    out = _final(num2, den2, b2, batch, g1W, g1b, g2W, g2b, l1W, l1b,
                 l2W, l2b)
    return out.reshape(-1)


# speedup vs baseline: 137.1925x; 1.0076x over previous
import functools

import jax
import jax.numpy as jnp
from jax import lax
from jax.experimental import pallas as pl
from jax.experimental.pallas import tpu as pltpu
from jax.experimental.pallas import tpu_sc as plsc

_L = 16
_C = 128
_NC = 2
_NS = 16
_F32 = jnp.float32


def _proj_body(x_ref, wl_ref, wr_ref, xl_ref, xr_ref):
    xb = x_ref[...]
    xl_ref[...] = jnp.dot(xb, wl_ref[...], preferred_element_type=_F32)
    xr_ref[...] = jnp.dot(xb, wr_ref[...], preferred_element_type=_F32)


def _project(x, wl, wr, blk=1000):
    n, k = x.shape
    d = wl.shape[1]
    grid = (n + blk - 1) // blk
    return pl.pallas_call(
        _proj_body,
        grid=(grid,),
        in_specs=[
            pl.BlockSpec((blk, k), lambda i: (i, 0)),
            pl.BlockSpec((k, d), lambda i: (0, 0)),
            pl.BlockSpec((k, d), lambda i: (0, 0)),
        ],
        out_specs=[
            pl.BlockSpec((blk, d), lambda i: (i, 0)),
            pl.BlockSpec((blk, d), lambda i: (i, 0)),
        ],
        out_shape=[
            jax.ShapeDtypeStruct((n, d), _F32),
            jax.ShapeDtypeStruct((n, d), _F32),
        ],
    )(x, wl, wr)


def _make_edge_kernel(n_nodes, d, heads, n_edges):
    ch = d // heads
    assert n_edges % _C == 0
    n_chunks = n_edges // _C
    nw = _NC * _NS
    jmax = n_chunks // nw
    nleft = n_chunks - jmax * nw
    assert jmax % 2 == 0 and jmax >= 4
    assert n_nodes % _NS == 0
    rps = (n_nodes // _NS) & ~7
    tail = n_nodes - rps * _NS
    assert tail % 8 == 0

    mesh = plsc.VectorSubcoreMesh(core_axis_name="c", subcore_axis_name="s")

    @functools.partial(
        pl.kernel,
        out_type=(
            jax.ShapeDtypeStruct((_NC, n_nodes, d), _F32),
            jax.ShapeDtypeStruct((_NC, n_nodes, heads), _F32),
        ),
        mesh=mesh,
        compiler_params=pltpu.CompilerParams(needs_layout_passes=False,
                                             use_tc_tiling_on_sc=False),
        scratch_types=[
            pltpu.VMEM((2, _C), jnp.int32),
            pltpu.VMEM((2, _C), jnp.int32),
            pltpu.VMEM((_C,), jnp.int32),
            pltpu.VMEM((_C,), jnp.int32),
            pltpu.VMEM((_C, d), _F32),
            pltpu.VMEM((_C, d), _F32),
            pltpu.VMEM((_C, d), _F32),
            pltpu.VMEM((_C, d), _F32),
            pltpu.VMEM((_C, d), _F32),
            pltpu.VMEM((_C, d), _F32),
            pltpu.VMEM((_C, heads), _F32),
            pltpu.VMEM((_C, heads), _F32),
            pltpu.VMEM((d,), _F32),
            pltpu.VMEM_SHARED((n_nodes, d), _F32),
            pltpu.VMEM_SHARED((n_nodes, heads), _F32),
            pltpu.SemaphoreType.DMA,
            pltpu.SemaphoreType.DMA,
            pltpu.SemaphoreType.DMA,
            pltpu.SemaphoreType.DMA,
            pltpu.SemaphoreType.DMA,
            pltpu.SemaphoreType.DMA,
        ],
    )
    def edge_kernel(edge_hbm, xl_hbm, xr_hbm, attb_hbm, znum_hbm,
                    zden_hbm, num_out, den_out, idx0, idx1, sidx0, sidx1,
                    xl0, xl1, xr0, xr1, sc0, sc1, pb0, pb1, att_v,
                    num_acc, den_acc, sem_i0, sem_i1, sem_g0, sem_g1,
                    sem_s0, sem_s1):
        cid = lax.axis_index("c")
        sid = lax.axis_index("s")
        wid = sid * _NC + cid
        iota = lax.iota(jnp.int32, _L)

        idx_v = (idx0, idx1)
        sidx = (sidx0, sidx1)
        xl_rows = (xl0, xl1)
        xr_rows = (xr0, xr1)
        scaled = (sc0, sc1)
        pbuf = (pb0, pb1)
        sem_i = (sem_i0, sem_i1)
        sem_g = (sem_g0, sem_g1)
        sem_s = (sem_s0, sem_s1)

        pltpu.sync_copy(attb_hbm, att_v)

        r0 = sid * rps
        pltpu.sync_copy(znum_hbm.at[pl.ds(0, rps)], num_acc.at[pl.ds(r0, rps)])
        pltpu.sync_copy(zden_hbm.at[pl.ds(0, rps)], den_acc.at[pl.ds(r0, rps)])
        if tail:
            @pl.when(sid == _NS - 1)
            def _():
                t0 = rps * _NS
                pltpu.sync_copy(znum_hbm.at[pl.ds(0, tail)],
                                num_acc.at[pl.ds(t0, tail)])
                pltpu.sync_copy(zden_hbm.at[pl.ds(0, tail)],
                                den_acc.at[pl.ds(t0, tail)])
        plsc.subcore_barrier()

        def chunk_base(j):
            return (wid + j * nw) * _C

        def issue_idx(s, j):
            return pltpu.async_copy(
                edge_hbm.at[:, pl.ds(chunk_base(j), _C)], idx_v[s], sem_i[s])

        def issue_gathers(s):
            g0 = pltpu.async_copy(xl_hbm.at[idx_v[s].at[0]], xl_rows[s],
                                  sem_g[s])
            g1 = pltpu.async_copy(xr_hbm.at[idx_v[s].at[1]], xr_rows[s],
                                  sem_g[s])
            return g0, g1

        def wait_gathers(s):
            pltpu.make_async_copy(xl_hbm.at[idx_v[s].at[0]], xl_rows[s],
                                  sem_g[s]).wait()
            pltpu.make_async_copy(xr_hbm.at[idx_v[s].at[1]], xr_rows[s],
                                  sem_g[s]).wait()

        def wait_idx(s, j):
            pltpu.make_async_copy(
                edge_hbm.at[:, pl.ds(chunk_base(j), _C)], idx_v[s],
                sem_i[s]).wait()

        def issue_scatters(s):
            pltpu.async_copy(scaled[s], num_acc.at[sidx[s]], sem_s[s],
                             add=True)
            pltpu.async_copy(pbuf[s], den_acc.at[sidx[s]], sem_s[s],
                             add=True)

        def wait_scatters(s):
            pltpu.make_async_copy(scaled[s], num_acc.at[sidx[s]],
                                  sem_s[s]).wait()
            pltpu.make_async_copy(pbuf[s], den_acc.at[sidx[s]],
                                  sem_s[s]).wait()

        def snapshot_dst(s):
            for t in range(_C // _L):
                sidx[s][pl.ds(t * _L, _L)] = idx_v[s][1, pl.ds(t * _L, _L)]

        nseg = d // _L
        sph = nseg // heads
        mask0 = iota == 0
        unroll = 4

        def compute(s):
            attv = [att_v[pl.ds(q * _L, _L)] for q in range(nseg)]

            @plsc.parallel_loop(0, _C // unroll)
            def _(eg):
                for k in range(unroll):
                    e = eg * unroll + k
                    th = [None] * heads
                    avals = []
                    for q in range(nseg):
                        a = xl_rows[s][e, pl.ds(q * _L, _L)]
                        b = xr_rows[s][e, pl.ds(q * _L, _L)]
                        m = a + b
                        m = jnp.where(m >= 0.0, m, 0.2 * m)
                        t = m * attv[q]
                        h = q // sph
                        th[h] = t if th[h] is None else th[h] + t
                        avals.append(a)
                    evec = jnp.full((_L,), e, jnp.int32)
                    for h in range(heads):
                        eh = jnp.sum(th[h])
                        pv = jnp.exp(jnp.full((_L,), eh, _F32))
                        plsc.store_scatter(
                            pbuf[s], [evec, jnp.full((_L,), h, jnp.int32)],
                            pv, mask=mask0)
                        for q in range(h * sph, (h + 1) * sph):
                            scaled[s][e, pl.ds(q * _L, _L)] = avals[q] * pv

        def slot_step(s, j, *, do_idx=True, do_next=True, do_waitsc=True):
            wait_gathers(s)
            if do_waitsc:
                wait_scatters(s)
            snapshot_dst(s)
            if do_idx:
                issue_idx(s, j + 2)
            if do_next:
                wait_idx(s ^ 1, j + 1)
                issue_gathers(s ^ 1)
            compute(s)
            issue_scatters(s)

        issue_idx(0, 0)
        wait_idx(0, 0)
        issue_gathers(0)
        issue_idx(1, 1)

        slot_step(0, 0, do_waitsc=False)
        slot_step(1, 1, do_waitsc=False)

        def pair_body(k, carry):
            j0 = 2 * k
            slot_step(0, j0)
            slot_step(1, j0 + 1)
            return carry

        lax.fori_loop(1, jmax // 2 - 1, pair_body, 0)

        slot_step(0, jmax - 2, do_idx=False)
        slot_step(1, jmax - 1, do_idx=False, do_next=False)
        wait_scatters(0)
        wait_scatters(1)

        if nleft:
            @pl.when(wid < nleft)
            def _():
                base = (jmax * nw + wid) * _C
                pltpu.sync_copy(edge_hbm.at[:, pl.ds(base, _C)], idx_v[0])
                g0, g1 = issue_gathers(0)
                g0.wait()
                g1.wait()
                snapshot_dst(0)
                compute(0)
                issue_scatters(0)
                wait_scatters(0)

        plsc.subcore_barrier()
        pltpu.sync_copy(num_acc.at[pl.ds(r0, rps)],
                        num_out.at[cid, pl.ds(r0, rps)])
        pltpu.sync_copy(den_acc.at[pl.ds(r0, rps)],
                        den_out.at[cid, pl.ds(r0, rps)])
        if tail:
            @pl.when(sid == _NS - 1)
            def _():
                t0 = rps * _NS
                pltpu.sync_copy(num_acc.at[pl.ds(t0, tail)],
                                num_out.at[cid, pl.ds(t0, tail)])
                pltpu.sync_copy(den_acc.at[pl.ds(t0, tail)],
                                den_out.at[cid, pl.ds(t0, tail)])

    return edge_kernel


def _edge_sweep(xl, xr, edge_index, att):
    n, d = xl.shape
    heads = att.shape[0]
    e = edge_index.shape[1]
    attb = att.reshape(d)
    znum = jnp.zeros((n // _NS, d), _F32)
    zden = jnp.zeros((n // _NS, heads), _F32)
    k = _make_edge_kernel(n, d, heads, e)
    num, den = k(edge_index, xl, xr, attb, znum, zden)
    return num, den


def _make_norm_body(heads, ch):
    def body(num_ref, den_ref, b_ref, wl_ref, wr_ref, xl_ref, xr_ref):
        n = num_ref[0] + num_ref[1]
        dsum = den_ref[0] + den_ref[1]
        parts = [
            n[:, h * ch:(h + 1) * ch] / (dsum[:, h:h + 1] + 1e-16)
            for h in range(heads)
        ]
        hcat = parts[0] if heads == 1 else jnp.concatenate(parts, axis=1)
        h1 = jnp.maximum(hcat + b_ref[...], 0.0)
        xl_ref[...] = jnp.dot(h1, wl_ref[...], preferred_element_type=_F32)
        xr_ref[...] = jnp.dot(h1, wr_ref[...], preferred_element_type=_F32)

    return body


def _norm_proj(num, den, b, wl, wr, blk=1000):
    _, n, d = num.shape
    heads = den.shape[2]
    d2 = wl.shape[1]
    grid = (n + blk - 1) // blk
    return pl.pallas_call(
        _make_norm_body(heads, d // heads),
        grid=(grid,),
        in_specs=[
            pl.BlockSpec((_NC, blk, d), lambda i: (0, i, 0)),
            pl.BlockSpec((_NC, blk, heads), lambda i: (0, i, 0)),
            pl.BlockSpec((1, d), lambda i: (0, 0)),
            pl.BlockSpec((d, d2), lambda i: (0, 0)),
            pl.BlockSpec((d, d2), lambda i: (0, 0)),
        ],
        out_specs=[
            pl.BlockSpec((blk, d2), lambda i: (i, 0)),
            pl.BlockSpec((blk, d2), lambda i: (i, 0)),
        ],
        out_shape=[
            jax.ShapeDtypeStruct((n, d2), _F32),
            jax.ShapeDtypeStruct((n, d2), _F32),
        ],
    )(num, den, b.reshape(1, d), wl, wr)


def _make_final_body(num_graphs):
    def body(num_ref, den_ref, b2_ref, batch_ref, g1w_ref, g1b_ref, g2w_ref,
             g2b_ref, l1w_ref, l1b_ref, l2w_ref, l2b_ref, out_ref):
        n = num_ref[0] + num_ref[1]
        dsum = den_ref[0] + den_ref[1]
        h2 = jnp.maximum(n / (dsum + 1e-16) + b2_ref[...], 0.0)
        z1 = jnp.maximum(
            jnp.dot(h2, g1w_ref[...], preferred_element_type=_F32)
            + g1b_ref[...], 0.0)
        gate = jnp.dot(z1, g2w_ref[...], preferred_element_type=_F32) \
            + g2b_ref[...]
        gex = jnp.exp(gate)
        nn = h2.shape[0]
        seg = lax.broadcasted_iota(jnp.int32, (num_graphs, nn), 0)
        onehot = jnp.where(seg == batch_ref[...], 1.0, 0.0)
        pnum = jnp.dot(onehot, h2 * gex, preferred_element_type=_F32)
        gden = jnp.dot(onehot, gex, preferred_element_type=_F32)
        pooled = pnum / (gden + 1e-16)
        z = jnp.maximum(
            jnp.dot(pooled, l1w_ref[...], preferred_element_type=_F32)
            + l1b_ref[...], 0.0)
        out_ref[...] = jnp.dot(z, l2w_ref[...],
                               preferred_element_type=_F32) + l2b_ref[...]

    return body


def _final(num, den, b2, batch, g1w, g1b, g2w, g2b, l1w, l1b, l2w, l2b,
           num_graphs=64):
    _, n, d = num.shape
    return pl.pallas_call(
        _make_final_body(num_graphs),
        out_shape=jax.ShapeDtypeStruct((num_graphs, 1), _F32),
    )(num, den, b2.reshape(1, d), batch.reshape(1, n), g1w,
      g1b.reshape(1, d), g2w, g2b.reshape(1, 1), l1w, l1b.reshape(1, d),
      l2w, l2b.reshape(1, 1))


def kernel(x, edge_index, batch, Wl1, Wr1, att1, b1, Wl2, Wr2, att2, b2,
           g1W, g1b, g2W, g2b, l1W, l1b, l2W, l2b):
    xl1, xr1 = _project(x, Wl1, Wr1)
    num1, den1 = _edge_sweep(xl1, xr1, edge_index, att1)
    xl2, xr2 = _norm_proj(num1, den1, b1, Wl2, Wr2)
    num2, den2 = _edge_sweep(xl2, xr2, edge_index, att2)
    out = _final(num2, den2, b2, batch, g1W, g1b, g2W, g2b, l1W, l1b,
                 l2W, l2b)
    return out.reshape(-1)


# speedup vs baseline: 151.4567x; 1.1040x over previous
import functools

import jax
import jax.numpy as jnp
from jax import lax
from jax.experimental import pallas as pl
from jax.experimental.pallas import tpu as pltpu
from jax.experimental.pallas import tpu_sc as plsc

_L = 16
_C = 128
_NC = 2
_NS = 16
_F32 = jnp.float32


def _proj_body(x_ref, wl_ref, wr_ref, xl_ref, xr_ref):
    xb = x_ref[...]
    xl_ref[...] = jnp.dot(xb, wl_ref[...], preferred_element_type=_F32)
    xr_ref[...] = jnp.dot(xb, wr_ref[...], preferred_element_type=_F32)


def _project(x, wl, wr, blk=1000):
    n, k = x.shape
    d = wl.shape[1]
    grid = (n + blk - 1) // blk
    return pl.pallas_call(
        _proj_body,
        grid=(grid,),
        in_specs=[
            pl.BlockSpec((blk, k), lambda i: (i, 0)),
            pl.BlockSpec((k, d), lambda i: (0, 0)),
            pl.BlockSpec((k, d), lambda i: (0, 0)),
        ],
        out_specs=[
            pl.BlockSpec((blk, d), lambda i: (i, 0)),
            pl.BlockSpec((blk, d), lambda i: (i, 0)),
        ],
        out_shape=[
            jax.ShapeDtypeStruct((n, d), _F32),
            jax.ShapeDtypeStruct((n, d), _F32),
        ],
    )(x, wl, wr)


def _make_edge_kernel(n_nodes, d, heads, n_edges):
    ch = d // heads
    assert n_edges % _C == 0
    n_chunks = n_edges // _C
    nw = _NC * _NS
    jmax = n_chunks // nw
    nleft = n_chunks - jmax * nw
    assert jmax % 3 == 0 and jmax >= 9
    assert n_nodes % _NS == 0
    rps = (n_nodes // _NS) & ~7
    tail = n_nodes - rps * _NS
    assert tail % 8 == 0

    mesh = plsc.VectorSubcoreMesh(core_axis_name="c", subcore_axis_name="s")

    scratch = []
    for _slot in range(3):
        scratch += [
            pltpu.VMEM((2, _C), jnp.int32),
            pltpu.VMEM((_C,), jnp.int32),
            pltpu.VMEM((_C, d), _F32),
            pltpu.VMEM((_C, d), _F32),
            pltpu.VMEM((_C, d), _F32),
            pltpu.VMEM((_C, heads), _F32),
        ]
    scratch += [
        pltpu.VMEM((d,), _F32),
        pltpu.VMEM_SHARED((n_nodes, d), _F32),
        pltpu.VMEM_SHARED((n_nodes, heads), _F32),
    ]
    scratch += [pltpu.SemaphoreType.DMA] * 9

    @functools.partial(
        pl.kernel,
        out_type=(
            jax.ShapeDtypeStruct((_NC, n_nodes, d), _F32),
            jax.ShapeDtypeStruct((_NC, n_nodes, heads), _F32),
        ),
        mesh=mesh,
        compiler_params=pltpu.CompilerParams(needs_layout_passes=False,
                                             use_tc_tiling_on_sc=False),
        scratch_types=scratch,
    )
    def edge_kernel(edge_hbm, xl_hbm, xr_hbm, attb_hbm, znum_hbm,
                    zden_hbm, num_out, den_out,
                    idx0, sidx0, xl0, xr0, sc0, pb0,
                    idx1, sidx1, xl1, xr1, sc1, pb1,
                    idx2, sidx2, xl2, xr2, sc2, pb2,
                    att_v, num_acc, den_acc,
                    sem_i0, sem_i1, sem_i2, sem_g0, sem_g1, sem_g2,
                    sem_s0, sem_s1, sem_s2):
        cid = lax.axis_index("c")
        sid = lax.axis_index("s")
        wid = sid * _NC + cid
        iota = lax.iota(jnp.int32, _L)

        idx_v = (idx0, idx1, idx2)
        sidx = (sidx0, sidx1, sidx2)
        xl_rows = (xl0, xl1, xl2)
        xr_rows = (xr0, xr1, xr2)
        scaled = (sc0, sc1, sc2)
        pbuf = (pb0, pb1, pb2)
        sem_i = (sem_i0, sem_i1, sem_i2)
        sem_g = (sem_g0, sem_g1, sem_g2)
        sem_s = (sem_s0, sem_s1, sem_s2)

        pltpu.sync_copy(attb_hbm, att_v)

        r0 = sid * rps
        pltpu.sync_copy(znum_hbm.at[pl.ds(0, rps)], num_acc.at[pl.ds(r0, rps)])
        pltpu.sync_copy(zden_hbm.at[pl.ds(0, rps)], den_acc.at[pl.ds(r0, rps)])
        if tail:
            @pl.when(sid == _NS - 1)
            def _():
                t0 = rps * _NS
                pltpu.sync_copy(znum_hbm.at[pl.ds(0, tail)],
                                num_acc.at[pl.ds(t0, tail)])
                pltpu.sync_copy(zden_hbm.at[pl.ds(0, tail)],
                                den_acc.at[pl.ds(t0, tail)])
        plsc.subcore_barrier()

        def chunk_base(j):
            return (wid + j * nw) * _C

        def issue_idx(s, j):
            return pltpu.async_copy(
                edge_hbm.at[:, pl.ds(chunk_base(j), _C)], idx_v[s], sem_i[s])

        def issue_gathers(s):
            g0 = pltpu.async_copy(xl_hbm.at[idx_v[s].at[0]], xl_rows[s],
                                  sem_g[s])
            g1 = pltpu.async_copy(xr_hbm.at[idx_v[s].at[1]], xr_rows[s],
                                  sem_g[s])
            return g0, g1

        def wait_gathers(s):
            pltpu.make_async_copy(xl_hbm.at[idx_v[s].at[0]], xl_rows[s],
                                  sem_g[s]).wait()
            pltpu.make_async_copy(xr_hbm.at[idx_v[s].at[1]], xr_rows[s],
                                  sem_g[s]).wait()

        def wait_idx(s, j):
            pltpu.make_async_copy(
                edge_hbm.at[:, pl.ds(chunk_base(j), _C)], idx_v[s],
                sem_i[s]).wait()

        def issue_scatters(s):
            pltpu.async_copy(scaled[s], num_acc.at[sidx[s]], sem_s[s],
                             add=True)
            pltpu.async_copy(pbuf[s], den_acc.at[sidx[s]], sem_s[s],
                             add=True)

        def wait_scatters(s):
            pltpu.make_async_copy(scaled[s], num_acc.at[sidx[s]],
                                  sem_s[s]).wait()
            pltpu.make_async_copy(pbuf[s], den_acc.at[sidx[s]],
                                  sem_s[s]).wait()

        def snapshot_dst(s):
            for t in range(_C // _L):
                sidx[s][pl.ds(t * _L, _L)] = idx_v[s][1, pl.ds(t * _L, _L)]

        nseg = d // _L
        sph = nseg // heads
        mask0 = iota == 0
        unroll = 4

        def compute(s):
            attv = [att_v[pl.ds(q * _L, _L)] for q in range(nseg)]

            @plsc.parallel_loop(0, _C // unroll)
            def _(eg):
                for k in range(unroll):
                    e = eg * unroll + k
                    th = [None] * heads
                    avals = []
                    for q in range(nseg):
                        a = xl_rows[s][e, pl.ds(q * _L, _L)]
                        b = xr_rows[s][e, pl.ds(q * _L, _L)]
                        m = a + b
                        m = jnp.where(m >= 0.0, m, 0.2 * m)
                        t = m * attv[q]
                        h = q // sph
                        th[h] = t if th[h] is None else th[h] + t
                        avals.append(a)
                    evec = jnp.full((_L,), e, jnp.int32)
                    for h in range(heads):
                        eh = jnp.sum(th[h])
                        pv = jnp.exp(jnp.full((_L,), eh, _F32))
                        plsc.store_scatter(
                            pbuf[s], [evec, jnp.full((_L,), h, jnp.int32)],
                            pv, mask=mask0)
                        for q in range(h * sph, (h + 1) * sph):
                            scaled[s][e, pl.ds(q * _L, _L)] = avals[q] * pv

        def slot_step(s, j, *, do_g2=True, do_idx3=True, do_waitsc=True):
            wait_gathers(s)
            if do_waitsc:
                wait_scatters(s)
            snapshot_dst(s)
            if do_g2:
                s2 = (s + 2) % 3
                wait_idx(s2, j + 2)
                issue_gathers(s2)
            if do_idx3:
                issue_idx(s, j + 3)
            compute(s)
            issue_scatters(s)

        issue_idx(0, 0)
        wait_idx(0, 0)
        issue_gathers(0)
        issue_idx(1, 1)
        wait_idx(1, 1)
        issue_gathers(1)
        issue_idx(2, 2)

        slot_step(0, 0, do_waitsc=False)
        slot_step(1, 1, do_waitsc=False)
        slot_step(2, 2, do_waitsc=False)

        def triple_body(k, carry):
            j0 = 3 * k
            slot_step(0, j0)
            slot_step(1, j0 + 1)
            slot_step(2, j0 + 2)
            return carry

        lax.fori_loop(1, jmax // 3 - 1, triple_body, 0)

        slot_step(0, jmax - 3, do_idx3=False)
        slot_step(1, jmax - 2, do_g2=False, do_idx3=False)
        slot_step(2, jmax - 1, do_g2=False, do_idx3=False)
        wait_scatters(0)
        wait_scatters(1)
        wait_scatters(2)

        if nleft:
            @pl.when(wid < nleft)
            def _():
                base = (jmax * nw + wid) * _C
                pltpu.sync_copy(edge_hbm.at[:, pl.ds(base, _C)], idx_v[0])
                g0, g1 = issue_gathers(0)
                g0.wait()
                g1.wait()
                snapshot_dst(0)
                compute(0)
                issue_scatters(0)
                wait_scatters(0)

        plsc.subcore_barrier()
        pltpu.sync_copy(num_acc.at[pl.ds(r0, rps)],
                        num_out.at[cid, pl.ds(r0, rps)])
        pltpu.sync_copy(den_acc.at[pl.ds(r0, rps)],
                        den_out.at[cid, pl.ds(r0, rps)])
        if tail:
            @pl.when(sid == _NS - 1)
            def _():
                t0 = rps * _NS
                pltpu.sync_copy(num_acc.at[pl.ds(t0, tail)],
                                num_out.at[cid, pl.ds(t0, tail)])
                pltpu.sync_copy(den_acc.at[pl.ds(t0, tail)],
                                den_out.at[cid, pl.ds(t0, tail)])

    return edge_kernel


def _edge_sweep(xl, xr, edge_index, att):
    n, d = xl.shape
    heads = att.shape[0]
    e = edge_index.shape[1]
    attb = att.reshape(d)
    znum = jnp.zeros((n // _NS, d), _F32)
    zden = jnp.zeros((n // _NS, heads), _F32)
    k = _make_edge_kernel(n, d, heads, e)
    num, den = k(edge_index, xl, xr, attb, znum, zden)
    return num, den


def _make_norm_body(heads, ch):
    def body(num_ref, den_ref, b_ref, wl_ref, wr_ref, xl_ref, xr_ref):
        n = num_ref[0] + num_ref[1]
        dsum = den_ref[0] + den_ref[1]
        parts = [
            n[:, h * ch:(h + 1) * ch] / (dsum[:, h:h + 1] + 1e-16)
            for h in range(heads)
        ]
        hcat = parts[0] if heads == 1 else jnp.concatenate(parts, axis=1)
        h1 = jnp.maximum(hcat + b_ref[...], 0.0)
        xl_ref[...] = jnp.dot(h1, wl_ref[...], preferred_element_type=_F32)
        xr_ref[...] = jnp.dot(h1, wr_ref[...], preferred_element_type=_F32)

    return body


def _norm_proj(num, den, b, wl, wr, blk=1000):
    _, n, d = num.shape
    heads = den.shape[2]
    d2 = wl.shape[1]
    grid = (n + blk - 1) // blk
    return pl.pallas_call(
        _make_norm_body(heads, d // heads),
        grid=(grid,),
        in_specs=[
            pl.BlockSpec((_NC, blk, d), lambda i: (0, i, 0)),
            pl.BlockSpec((_NC, blk, heads), lambda i: (0, i, 0)),
            pl.BlockSpec((1, d), lambda i: (0, 0)),
            pl.BlockSpec((d, d2), lambda i: (0, 0)),
            pl.BlockSpec((d, d2), lambda i: (0, 0)),
        ],
        out_specs=[
            pl.BlockSpec((blk, d2), lambda i: (i, 0)),
            pl.BlockSpec((blk, d2), lambda i: (i, 0)),
        ],
        out_shape=[
            jax.ShapeDtypeStruct((n, d2), _F32),
            jax.ShapeDtypeStruct((n, d2), _F32),
        ],
    )(num, den, b.reshape(1, d), wl, wr)


def _make_final_body(num_graphs):
    def body(num_ref, den_ref, b2_ref, batch_ref, g1w_ref, g1b_ref, g2w_ref,
             g2b_ref, l1w_ref, l1b_ref, l2w_ref, l2b_ref, out_ref):
        n = num_ref[0] + num_ref[1]
        dsum = den_ref[0] + den_ref[1]
        h2 = jnp.maximum(n / (dsum + 1e-16) + b2_ref[...], 0.0)
        z1 = jnp.maximum(
            jnp.dot(h2, g1w_ref[...], preferred_element_type=_F32)
            + g1b_ref[...], 0.0)
        gate = jnp.dot(z1, g2w_ref[...], preferred_element_type=_F32) \
            + g2b_ref[...]
        gex = jnp.exp(gate)
        nn = h2.shape[0]
        seg = lax.broadcasted_iota(jnp.int32, (num_graphs, nn), 0)
        onehot = jnp.where(seg == batch_ref[...], 1.0, 0.0)
        pnum = jnp.dot(onehot, h2 * gex, preferred_element_type=_F32)
        gden = jnp.dot(onehot, gex, preferred_element_type=_F32)
        pooled = pnum / (gden + 1e-16)
        z = jnp.maximum(
            jnp.dot(pooled, l1w_ref[...], preferred_element_type=_F32)
            + l1b_ref[...], 0.0)
        out_ref[...] = jnp.dot(z, l2w_ref[...],
                               preferred_element_type=_F32) + l2b_ref[...]

    return body


def _final(num, den, b2, batch, g1w, g1b, g2w, g2b, l1w, l1b, l2w, l2b,
           num_graphs=64):
    _, n, d = num.shape
    return pl.pallas_call(
        _make_final_body(num_graphs),
        out_shape=jax.ShapeDtypeStruct((num_graphs, 1), _F32),
    )(num, den, b2.reshape(1, d), batch.reshape(1, n), g1w,
      g1b.reshape(1, d), g2w, g2b.reshape(1, 1), l1w, l1b.reshape(1, d),
      l2w, l2b.reshape(1, 1))


def kernel(x, edge_index, batch, Wl1, Wr1, att1, b1, Wl2, Wr2, att2, b2,
           g1W, g1b, g2W, g2b, l1W, l1b, l2W, l2b):
    xl1, xr1 = _project(x, Wl1, Wr1)
    num1, den1 = _edge_sweep(xl1, xr1, edge_index, att1)
    xl2, xr2 = _norm_proj(num1, den1, b1, Wl2, Wr2)
    num2, den2 = _edge_sweep(xl2, xr2, edge_index, att2)
    out = _final(num2, den2, b2, batch, g1W, g1b, g2W, g2b, l1W, l1b,
                 l2W, l2b)
    return out.reshape(-1)
